# baseline (speedup 1.0000x reference)
import numpy as np
import jax
import jax.numpy as jnp
from jax.experimental import pallas as pl
from jax.experimental.pallas import tpu as pltpu

_L = 50
_K = 10
_D = 256
_OFFSETS = tuple(d for d in range(-_K, _K + 1) if d != 0)


def _bsilu(x):
    xb = x.astype(jnp.bfloat16)
    half = jnp.bfloat16(0.5)
    return xb * (half * jnp.tanh(half * xb) + half)


def _bdot(x, w):
    return jnp.dot(x.astype(jnp.bfloat16), w, preferred_element_type=jnp.float32)


def _bdot16(x, w):
    return _bdot(x, w).astype(jnp.bfloat16)


def _fwd_body(sb, refs):
    n = sb * _L
    (coords_ref, pos_ref, lseq_ref, wn_ref, bn_ref) = refs[:5]
    layer_refs = refs[5 : 5 + 3 * 14]
    (lng_ref, lnb_ref, wp_ref, bp_ref, out_ref) = refs[5 + 3 * 14 :]

    c = coords_ref[...]
    h = jnp.dot(c, wn_ref[...], preferred_element_type=jnp.float32) + bn_ref[...]
    pos = pos_ref[...]

    zpad_d = jnp.zeros((_K, _D), jnp.bfloat16)
    zpad_c = jnp.zeros((_K, 4), jnp.float32)

    for l in range(3):
        (w1a, w1b, w1c, b1, w2, b2, wc1, bc1, wc2,
         wn1a, wn1b, bn1, wn2, bn2) = [r[...] for r in layer_refs[l * 14 : (l + 1) * 14]]
        a_row = ((_bdot(h, w1a) + b1)).astype(jnp.bfloat16)
        b_col = _bdot16(h, w1b)
        b_pad = jnp.concatenate([zpad_d, b_col, zpad_d], axis=0)
        c_pad = jnp.concatenate([zpad_c, c, zpad_c], axis=0)

        agg = jnp.zeros((n, _D), jnp.float32)
        upd = jnp.zeros((n, 4), jnp.float32)
        for d in _OFFSETS:
            csh = jax.lax.slice(c_pad, (_K + d, 0), (_K + d + n, 4))
            diff = c - csh
            radial = jnp.sum(diff * diff, axis=1, keepdims=True)
            bsh = jax.lax.slice(b_pad, (_K + d, 0), (_K + d + n, _D))
            pre = a_row + bsh + radial.astype(jnp.bfloat16) * w1c
            msg = _bsilu(pre)
            msg = _bsilu(_bdot16(msg, w2) + b2)
            cw = _bdot(_bsilu(_bdot16(msg, wc1) + bc1), wc2)
            valid = (pos + d >= 0) & (pos + d < _L)
            vf = valid.astype(jnp.float32)
            agg = agg + msg.astype(jnp.float32) * vf
            upd = upd + diff * (cw * vf)
        c = c + upd
        hn = _bsilu(_bdot16(h, wn1a) + _bdot16(agg, wn1b) + bn1)
        h = _bdot(hn, wn2) + bn2 + h

    mu = jnp.mean(h, axis=1, keepdims=True)
    xc = h - mu
    var = jnp.mean(xc * xc, axis=1, keepdims=True)
    hN = xc * jax.lax.rsqrt(var + 1e-5) * lng_ref[...] + lnb_ref[...]
    score = jnp.dot(hN, wp_ref[...], preferred_element_type=jnp.float32) + bp_ref[...]
    e = jnp.exp(score - jnp.max(score))
    lseq = lseq_ref[...]
    seg = (lseq == jax.lax.broadcasted_iota(jnp.int32, (1, sb), 1)).astype(jnp.float32)
    colsum = jax.lax.dot_general(seg, e, (((0,), (0,)), ((), ())),
                                 preferred_element_type=jnp.float32)
    denom = jnp.dot(seg, colsum, preferred_element_type=jnp.float32)
    w = e / denom
    out_ref[...] = jax.lax.dot_general(seg * w, hN, (((0,), (0,)), ((), ())),
                                       preferred_element_type=jnp.float32)


def kernel(coords_batch, batch_idx, edge_index_3d_list, params):
    nb = batch_idx.shape[0] // _L
    sb = 8 if nb % 8 == 0 else 1
    nblk = nb // sb
    n_blk = sb * _L
    coords4 = jnp.pad(coords_batch.astype(jnp.float32), ((0, 0), (0, 1)))
    pos = jnp.asarray((np.arange(n_blk, dtype=np.int32) % _L).reshape(n_blk, 1))
    lseq = jnp.asarray((np.arange(n_blk, dtype=np.int32) // _L).reshape(n_blk, 1))
    wn = jnp.pad(params["node_proj"]["W"], ((0, 1), (0, 0)))
    bn = params["node_proj"]["b"].reshape(1, _D)

    ops = [coords4, pos, lseq, wn, bn]
    bf16 = jnp.bfloat16
    for lp in params["layers"]:
        w1 = lp["edge1"]["W"]
        ops += [
            w1[:_D].astype(bf16), w1[_D:2 * _D].astype(bf16),
            w1[2 * _D:].reshape(1, _D).astype(bf16),
            lp["edge1"]["b"].reshape(1, _D),
            lp["edge2"]["W"].astype(bf16),
            lp["edge2"]["b"].reshape(1, _D).astype(bf16),
            lp["coord1"]["W"].astype(bf16),
            lp["coord1"]["b"].reshape(1, _D).astype(bf16),
            lp["coord2"]["W"].astype(bf16),
            lp["node1"]["W"][:_D].astype(bf16), lp["node1"]["W"][_D:].astype(bf16),
            lp["node1"]["b"].reshape(1, _D).astype(bf16),
            lp["node2"]["W"].astype(bf16), lp["node2"]["b"].reshape(1, _D),
        ]
    ops += [
        params["ln_g"].reshape(1, _D), params["ln_b"].reshape(1, _D),
        params["pool"]["W"], params["pool"]["b"].reshape(1, 1),
    ]

    def const_spec(arr):
        return pl.BlockSpec(arr.shape, lambda i: (0, 0))

    in_specs = [pl.BlockSpec((n_blk, 4), lambda i: (i, 0))]
    in_specs += [const_spec(a) for a in ops[1:]]

    body = lambda *refs: _fwd_body(sb, refs)
    return pl.pallas_call(
        body,
        grid=(nblk,),
        compiler_params=pltpu.CompilerParams(
            dimension_semantics=("parallel",)),
        in_specs=in_specs,
        out_specs=pl.BlockSpec((sb, _D), lambda i: (i, 0)),
        out_shape=jax.ShapeDtypeStruct((nb, _D), jnp.float32),
    )(*ops)


# speedup vs baseline: 1.0309x; 1.0309x over previous
import numpy as np
import jax
import jax.numpy as jnp
from jax.experimental import pallas as pl
from jax.experimental.pallas import tpu as pltpu

_L = 50
_K = 10
_D = 256
_OFFSETS = tuple(d for d in range(-_K, _K + 1) if d != 0)


def _bsilu(x):
    xb = x.astype(jnp.bfloat16)
    half = jnp.bfloat16(0.5)
    return xb * (half * jnp.tanh(half * xb) + half)


def _bdot(x, w):
    return jnp.dot(x.astype(jnp.bfloat16), w, preferred_element_type=jnp.float32)


def _bdot16(x, w):
    return _bdot(x, w).astype(jnp.bfloat16)


def _fwd_body(sb, refs):
    n = sb * _L
    (coords_ref, lseq_ref, wn_ref, bn_ref) = refs[:4]
    vf_refs = refs[4 : 4 + len(_OFFSETS)]
    vfb_refs = refs[4 + len(_OFFSETS) : 4 + 2 * len(_OFFSETS)]
    base = 4 + 2 * len(_OFFSETS)
    layer_refs = refs[base : base + 3 * 14]
    (lng_ref, lnb_ref, wp_ref, bp_ref, out_ref) = refs[base + 3 * 14 :]

    c = coords_ref[...]
    h = jnp.dot(c, wn_ref[...], preferred_element_type=jnp.float32) + bn_ref[...]

    zpad_d = jnp.zeros((_K, _D), jnp.bfloat16)
    zpad_c = jnp.zeros((_K, 4), jnp.float32)

    for l in range(3):
        (w1a, w1b, w1c, b1, w2, b2, wc1, bc1, wc2,
         wn1a, wn1b, bn1, wn2, bn2) = [r[...] for r in layer_refs[l * 14 : (l + 1) * 14]]
        a_row = ((_bdot(h, w1a) + b1)).astype(jnp.bfloat16)
        b_col = _bdot16(h, w1b)
        b_pad = jnp.concatenate([zpad_d, b_col, zpad_d], axis=0)
        c_pad = jnp.concatenate([zpad_c, c, zpad_c], axis=0)

        agg = jnp.zeros((n, _D), jnp.float32)
        upd = jnp.zeros((n, 4), jnp.float32)
        pending = None
        for k, d in enumerate(_OFFSETS):
            csh = jax.lax.slice(c_pad, (_K + d, 0), (_K + d + n, 4))
            diff = c - csh
            radial = jnp.sum(diff * diff, axis=1, keepdims=True)
            bsh = jax.lax.slice(b_pad, (_K + d, 0), (_K + d + n, _D))
            pre = a_row + bsh + radial.astype(jnp.bfloat16) * w1c
            msg = _bsilu(pre)
            msg = _bsilu(_bdot16(msg, w2) + b2)
            cw = _bdot(_bsilu(_bdot16(msg, wc1) + bc1), wc2)
            mm = msg * vfb_refs[k][...]
            if pending is None:
                pending = mm
            else:
                agg = agg + (pending + mm).astype(jnp.float32)
                pending = None
            upd = upd + diff * (cw * vf_refs[k][...])
        if pending is not None:
            agg = agg + pending.astype(jnp.float32)
        c = c + upd
        hn = _bsilu(_bdot16(h, wn1a) + _bdot16(agg, wn1b) + bn1)
        h = _bdot(hn, wn2) + bn2 + h

    mu = jnp.mean(h, axis=1, keepdims=True)
    xc = h - mu
    var = jnp.mean(xc * xc, axis=1, keepdims=True)
    hN = xc * jax.lax.rsqrt(var + 1e-5) * lng_ref[...] + lnb_ref[...]
    score = jnp.dot(hN, wp_ref[...], preferred_element_type=jnp.float32) + bp_ref[...]
    e = jnp.exp(score - jnp.max(score))
    lseq = lseq_ref[...]
    seg = (lseq == jax.lax.broadcasted_iota(jnp.int32, (1, sb), 1)).astype(jnp.float32)
    colsum = jax.lax.dot_general(seg, e, (((0,), (0,)), ((), ())),
                                 preferred_element_type=jnp.float32)
    denom = jnp.dot(seg, colsum, preferred_element_type=jnp.float32)
    w = e / denom
    out_ref[...] = jax.lax.dot_general(seg * w, hN, (((0,), (0,)), ((), ())),
                                       preferred_element_type=jnp.float32)


def kernel(coords_batch, batch_idx, edge_index_3d_list, params):
    nb = batch_idx.shape[0] // _L
    sb = 8 if nb % 8 == 0 else 1
    nblk = nb // sb
    n_blk = sb * _L
    coords4 = jnp.pad(coords_batch.astype(jnp.float32), ((0, 0), (0, 1)))
    pos_np = (np.arange(n_blk, dtype=np.int32) % _L).reshape(n_blk, 1)
    lseq = jnp.asarray((np.arange(n_blk, dtype=np.int32) // _L).reshape(n_blk, 1))
    wn = jnp.pad(params["node_proj"]["W"], ((0, 1), (0, 0)))
    bn = params["node_proj"]["b"].reshape(1, _D)

    masks_np = [((pos_np + d >= 0) & (pos_np + d < _L)).astype(np.float32)
                for d in _OFFSETS]
    ops = [coords4, lseq, wn, bn]
    ops += [jnp.asarray(m) for m in masks_np]
    ops += [jnp.asarray(m).astype(jnp.bfloat16) for m in masks_np]
    bf16 = jnp.bfloat16
    for lp in params["layers"]:
        w1 = lp["edge1"]["W"]
        ops += [
            w1[:_D].astype(bf16), w1[_D:2 * _D].astype(bf16),
            w1[2 * _D:].reshape(1, _D).astype(bf16),
            lp["edge1"]["b"].reshape(1, _D),
            lp["edge2"]["W"].astype(bf16),
            lp["edge2"]["b"].reshape(1, _D).astype(bf16),
            lp["coord1"]["W"].astype(bf16),
            lp["coord1"]["b"].reshape(1, _D).astype(bf16),
            lp["coord2"]["W"].astype(bf16),
            lp["node1"]["W"][:_D].astype(bf16), lp["node1"]["W"][_D:].astype(bf16),
            lp["node1"]["b"].reshape(1, _D).astype(bf16),
            lp["node2"]["W"].astype(bf16), lp["node2"]["b"].reshape(1, _D),
        ]
    ops += [
        params["ln_g"].reshape(1, _D), params["ln_b"].reshape(1, _D),
        params["pool"]["W"], params["pool"]["b"].reshape(1, 1),
    ]

    def const_spec(arr):
        return pl.BlockSpec(arr.shape, lambda i: (0, 0))

    in_specs = [pl.BlockSpec((n_blk, 4), lambda i: (i, 0))]
    in_specs += [const_spec(a) for a in ops[1:]]

    body = lambda *refs: _fwd_body(sb, refs)
    return pl.pallas_call(
        body,
        grid=(nblk,),
        compiler_params=pltpu.CompilerParams(
            dimension_semantics=("parallel",)),
        in_specs=in_specs,
        out_specs=pl.BlockSpec((sb, _D), lambda i: (i, 0)),
        out_shape=jax.ShapeDtypeStruct((nb, _D), jnp.float32),
    )(*ops)


# speedup vs baseline: 1.0894x; 1.0567x over previous
import numpy as np
import jax
import jax.numpy as jnp
from jax.experimental import pallas as pl
from jax.experimental.pallas import tpu as pltpu

_L = 50
_K = 10
_D = 256
_OFFSETS = tuple(d for d in range(-_K, _K + 1) if d != 0)


def _bsilu_h(y):
    yb = y.astype(jnp.bfloat16)
    return yb * jnp.tanh(yb) + yb


def _bdot(x, w):
    return jnp.dot(x.astype(jnp.bfloat16), w, preferred_element_type=jnp.float32)


def _bdot16(x, w):
    return _bdot(x, w).astype(jnp.bfloat16)


def _fwd_body(sb, refs):
    n = sb * _L
    (coords_ref, lseq_ref, wn_ref, bn_ref) = refs[:4]
    vf_refs = refs[4 : 4 + len(_OFFSETS)]
    vfb_refs = refs[4 + len(_OFFSETS) : 4 + 2 * len(_OFFSETS)]
    base = 4 + 2 * len(_OFFSETS)
    layer_refs = refs[base : base + 3 * 14]
    (lng_ref, lnb_ref, wp_ref, bp_ref, out_ref) = refs[base + 3 * 14 :]

    c = coords_ref[...]
    h = jnp.dot(c, wn_ref[...], preferred_element_type=jnp.float32) + bn_ref[...]

    zpad_d = jnp.zeros((_K, _D), jnp.bfloat16)
    zpad_c = jnp.zeros((_K, 4), jnp.float32)

    for l in range(3):
        (w1a, w1b, w1c, b1, w2, b2, wc1, bc1, wc2,
         wn1a, wn1b, bn1, wn2, bn2) = [r[...] for r in layer_refs[l * 14 : (l + 1) * 14]]
        a_row = ((_bdot(h, w1a) + b1)).astype(jnp.bfloat16)
        b_col = _bdot16(h, w1b)
        b_pad = jnp.concatenate([zpad_d, b_col, zpad_d], axis=0)
        c_pad = jnp.concatenate([zpad_c, c, zpad_c], axis=0)

        agg = jnp.zeros((n, _D), jnp.float32)
        upd = jnp.zeros((n, 4), jnp.float32)
        pending = None
        for k, d in enumerate(_OFFSETS):
            csh = jax.lax.slice(c_pad, (_K + d, 0), (_K + d + n, 4))
            diff = c - csh
            radial = jnp.sum(diff * diff, axis=1, keepdims=True)
            bsh = jax.lax.slice(b_pad, (_K + d, 0), (_K + d + n, _D))
            pre = a_row + bsh + radial.astype(jnp.bfloat16) * w1c
            msg = _bsilu_h(pre)
            msg = _bsilu_h(_bdot16(msg, w2) + b2)
            cw = _bdot(_bsilu_h(_bdot16(msg, wc1) + bc1), wc2)
            mm = msg * vfb_refs[k][...]
            if pending is None:
                pending = mm
            else:
                agg = agg + (pending + mm).astype(jnp.float32)
                pending = None
            upd = upd + diff * (cw * vf_refs[k][...])
        if pending is not None:
            agg = agg + pending.astype(jnp.float32)
        c = c + upd
        hn = _bsilu_h(_bdot16(h, wn1a) + _bdot16(agg, wn1b) + bn1)
        h = _bdot(hn, wn2) + bn2 + h

    mu = jnp.mean(h, axis=1, keepdims=True)
    xc = h - mu
    var = jnp.mean(xc * xc, axis=1, keepdims=True)
    hN = xc * jax.lax.rsqrt(var + 1e-5) * lng_ref[...] + lnb_ref[...]
    score = jnp.dot(hN, wp_ref[...], preferred_element_type=jnp.float32) + bp_ref[...]
    e = jnp.exp(score - jnp.max(score))
    lseq = lseq_ref[...]
    seg = (lseq == jax.lax.broadcasted_iota(jnp.int32, (1, sb), 1)).astype(jnp.float32)
    colsum = jax.lax.dot_general(seg, e, (((0,), (0,)), ((), ())),
                                 preferred_element_type=jnp.float32)
    denom = jnp.dot(seg, colsum, preferred_element_type=jnp.float32)
    w = e / denom
    out_ref[...] = jax.lax.dot_general(seg * w, hN, (((0,), (0,)), ((), ())),
                                       preferred_element_type=jnp.float32)


def kernel(coords_batch, batch_idx, edge_index_3d_list, params):
    nb = batch_idx.shape[0] // _L
    sb = 8 if nb % 8 == 0 else 1
    nblk = nb // sb
    n_blk = sb * _L
    coords4 = jnp.pad(coords_batch.astype(jnp.float32), ((0, 0), (0, 1)))
    pos_np = (np.arange(n_blk, dtype=np.int32) % _L).reshape(n_blk, 1)
    lseq = jnp.asarray((np.arange(n_blk, dtype=np.int32) // _L).reshape(n_blk, 1))
    wn = jnp.pad(params["node_proj"]["W"], ((0, 1), (0, 0)))
    bn = params["node_proj"]["b"].reshape(1, _D)

    masks_np = [((pos_np + d >= 0) & (pos_np + d < _L)).astype(np.float32)
                for d in _OFFSETS]
    ops = [coords4, lseq, wn, bn]
    ops += [jnp.asarray(m) for m in masks_np]
    ops += [jnp.asarray(m).astype(jnp.bfloat16) for m in masks_np]
    bf16 = jnp.bfloat16
    for lp in params["layers"]:
        w1 = lp["edge1"]["W"] * 0.5
        ops += [
            w1[:_D].astype(bf16), w1[_D:2 * _D].astype(bf16),
            w1[2 * _D:].reshape(1, _D).astype(bf16),
            (lp["edge1"]["b"] * 0.5).reshape(1, _D),
            (lp["edge2"]["W"] * 0.5).astype(bf16),
            (lp["edge2"]["b"] * 0.5).reshape(1, _D).astype(bf16),
            (lp["coord1"]["W"] * 0.5).astype(bf16),
            (lp["coord1"]["b"] * 0.5).reshape(1, _D).astype(bf16),
            lp["coord2"]["W"].astype(bf16),
            (lp["node1"]["W"][:_D] * 0.5).astype(bf16),
            (lp["node1"]["W"][_D:] * 0.5).astype(bf16),
            (lp["node1"]["b"] * 0.5).reshape(1, _D).astype(bf16),
            lp["node2"]["W"].astype(bf16), lp["node2"]["b"].reshape(1, _D),
        ]
    ops += [
        params["ln_g"].reshape(1, _D), params["ln_b"].reshape(1, _D),
        params["pool"]["W"], params["pool"]["b"].reshape(1, 1),
    ]

    def const_spec(arr):
        return pl.BlockSpec(arr.shape, lambda i: (0, 0))

    in_specs = [pl.BlockSpec((n_blk, 4), lambda i: (i, 0))]
    in_specs += [const_spec(a) for a in ops[1:]]

    body = lambda *refs: _fwd_body(sb, refs)
    return pl.pallas_call(
        body,
        grid=(nblk,),
        compiler_params=pltpu.CompilerParams(
            dimension_semantics=("parallel",)),
        in_specs=in_specs,
        out_specs=pl.BlockSpec((sb, _D), lambda i: (i, 0)),
        out_shape=jax.ShapeDtypeStruct((nb, _D), jnp.float32),
    )(*ops)
